# initial kernel scaffold (unmeasured)
import jax
import jax.numpy as jnp
from jax import lax
from jax.experimental import pallas as pl
from jax.experimental.pallas import tpu as pltpu

N_DEV = 32
M, K, N = 4096, 4096, 8192
KS = K // N_DEV
NT = 256
N_TILES = N // NT


def kernel(x, w_mat, scale_x, scale_w):
    x8 = x.astype(jnp.float8_e5m2)
    w8 = w_mat.astype(jnp.float8_e5m2)
    scale = (scale_x[0] * scale_w[0]).reshape(1, 1)

    def body(x_ref, w_ref, s_ref, out_ref, xg, wg, acc,
             xs_send, xs_recv, ws_send, ws_recv, out_sem):
        my = lax.axis_index("i")
        right = lax.rem(my + 1, N_DEV)
        left = lax.rem(my + N_DEV - 1, N_DEV)

        barrier = pltpu.get_barrier_semaphore()
        for nbr in (left, right):
            pl.semaphore_signal(barrier, inc=1, device_id=(nbr,),
                                device_id_type=pl.DeviceIdType.MESH)
        pl.semaphore_wait(barrier, 2)

        xg[:, pl.ds(my * KS, KS)] = x_ref[...]
        wg[pl.ds(my * KS, KS), :] = w_ref[...]

        for h in range(N_DEV - 1):
            s = lax.rem(my - h + N_DEV, N_DEV)
            rx = pltpu.make_async_remote_copy(
                src_ref=xg.at[:, pl.ds(s * KS, KS)],
                dst_ref=xg.at[:, pl.ds(s * KS, KS)],
                send_sem=xs_send.at[h], recv_sem=xs_recv.at[h],
                device_id=(right,), device_id_type=pl.DeviceIdType.MESH)
            rw = pltpu.make_async_remote_copy(
                src_ref=wg.at[pl.ds(s * KS, KS), :],
                dst_ref=wg.at[pl.ds(s * KS, KS), :],
                send_sem=ws_send.at[h], recv_sem=ws_recv.at[h],
                device_id=(right,), device_id_type=pl.DeviceIdType.MESH)
            rx.start()
            rw.start()
            rx.wait()
            rw.wait()

        sc = s_ref[0, 0]
        for t in range(N_TILES):
            a = jnp.dot(xg[...], wg[:, t * NT:(t + 1) * NT],
                        preferred_element_type=jnp.float32)
            y = a * sc
            acc[...] = y * (1.0 / (1.0 + jnp.exp(-y)))
            cp = pltpu.make_async_copy(
                acc, out_ref.at[:, pl.ds(t * NT, NT)], out_sem)
            cp.start()
            cp.wait()

    return pl.pallas_call(
        body,
        out_shape=jax.ShapeDtypeStruct((M, N), jnp.float32),
        in_specs=[
            pl.BlockSpec(memory_space=pltpu.VMEM),
            pl.BlockSpec(memory_space=pltpu.VMEM),
            pl.BlockSpec(memory_space=pltpu.SMEM),
        ],
        out_specs=pl.BlockSpec(memory_space=pltpu.ANY),
        scratch_shapes=[
            pltpu.VMEM((M, K), jnp.float8_e5m2),
            pltpu.VMEM((K, N), jnp.float8_e5m2),
            pltpu.VMEM((M, NT), jnp.float32),
            pltpu.SemaphoreType.DMA((N_DEV - 1,)),
            pltpu.SemaphoreType.DMA((N_DEV - 1,)),
            pltpu.SemaphoreType.DMA((N_DEV - 1,)),
            pltpu.SemaphoreType.DMA((N_DEV - 1,)),
            pltpu.SemaphoreType.DMA,
        ],
        compiler_params=pltpu.CompilerParams(collective_id=0),
    )(x8, w8, scale)


# baseline (device time: 978381 ns/iter reference)
import jax
import jax.numpy as jnp
from jax import lax
from jax.experimental import pallas as pl
from jax.experimental.pallas import tpu as pltpu

N_DEV = 32
M, K, N = 4096, 4096, 8192
KS = K // N_DEV
NT = 256
N_TILES = N // NT
MT = 512
M_TILES = M // MT


def kernel(x, w_mat, scale_x, scale_w):
    x8 = x.astype(jnp.float8_e5m2)
    w8 = w_mat.astype(jnp.float8_e5m2)
    scale = (scale_x[0] * scale_w[0]).reshape(1, 1)

    def body(x_ref, w_ref, s_ref, out_ref, xg, wg, acc,
             xs_send, xs_recv, ws_send, ws_recv, out_sem):
        my = lax.axis_index("i")
        right = lax.rem(my + 1, N_DEV)
        left = lax.rem(my + N_DEV - 1, N_DEV)

        barrier = pltpu.get_barrier_semaphore()
        for nbr in (left, right):
            pl.semaphore_signal(barrier, inc=1, device_id=(nbr,),
                                device_id_type=pl.DeviceIdType.MESH)
        pl.semaphore_wait(barrier, 2)

        xg[:, pl.ds(my * KS, KS)] = x_ref[...]
        wg[pl.ds(my * KS, KS), :] = w_ref[...]

        def hop(h, carry):
            s = lax.rem(my - h + N_DEV, N_DEV)
            rx = pltpu.make_async_remote_copy(
                src_ref=xg.at[:, pl.ds(s * KS, KS)],
                dst_ref=xg.at[:, pl.ds(s * KS, KS)],
                send_sem=xs_send.at[h], recv_sem=xs_recv.at[h],
                device_id=(right,), device_id_type=pl.DeviceIdType.MESH)
            rw = pltpu.make_async_remote_copy(
                src_ref=wg.at[pl.ds(s * KS, KS), :],
                dst_ref=wg.at[pl.ds(s * KS, KS), :],
                send_sem=ws_send.at[h], recv_sem=ws_recv.at[h],
                device_id=(right,), device_id_type=pl.DeviceIdType.MESH)
            rx.start()
            rw.start()
            rx.wait()
            rw.wait()
            return carry

        lax.fori_loop(0, N_DEV - 1, hop, 0)

        sc = s_ref[0, 0]

        def tile(t, carry):
            w_t = wg[:, pl.ds(t * NT, NT)]

            def mtile(m, carry2):
                a = jnp.dot(xg[pl.ds(m * MT, MT), :], w_t,
                            preferred_element_type=jnp.float32)
                y = a * sc
                acc[pl.ds(m * MT, MT), :] = y * (1.0 / (1.0 + jnp.exp(-y)))
                return carry2

            lax.fori_loop(0, M_TILES, mtile, 0)
            cp = pltpu.make_async_copy(
                acc, out_ref.at[:, pl.ds(t * NT, NT)], out_sem)
            cp.start()
            cp.wait()
            return carry

        lax.fori_loop(0, N_TILES, tile, 0)

    return pl.pallas_call(
        body,
        out_shape=jax.ShapeDtypeStruct((M, N), jnp.float32),
        in_specs=[
            pl.BlockSpec(memory_space=pltpu.VMEM),
            pl.BlockSpec(memory_space=pltpu.VMEM),
            pl.BlockSpec(memory_space=pltpu.SMEM),
        ],
        out_specs=pl.BlockSpec(memory_space=pltpu.MemorySpace.HBM),
        scratch_shapes=[
            pltpu.VMEM((M, K), jnp.float8_e5m2),
            pltpu.VMEM((K, N), jnp.float8_e5m2),
            pltpu.VMEM((M, NT), jnp.float32),
            pltpu.SemaphoreType.DMA((N_DEV - 1,)),
            pltpu.SemaphoreType.DMA((N_DEV - 1,)),
            pltpu.SemaphoreType.DMA((N_DEV - 1,)),
            pltpu.SemaphoreType.DMA((N_DEV - 1,)),
            pltpu.SemaphoreType.DMA,
        ],
        compiler_params=pltpu.CompilerParams(
            collective_id=0,
            vmem_limit_bytes=100 * 1024 * 1024,
        ),
    )(x8, w8, scale)


# device time: 895746 ns/iter; 1.0923x vs baseline; 1.0923x over previous
import jax
import jax.numpy as jnp
from jax import lax
from jax.experimental import pallas as pl
from jax.experimental.pallas import tpu as pltpu

N_DEV = 32
M, K, N = 4096, 4096, 8192
KS = K // N_DEV
NT = 256
N_TILES = N // NT
MT = 512
M_TILES = M // MT
H_RIGHT = N_DEV // 2
H_LEFT = N_DEV // 2 - 1


def kernel(x, w_mat, scale_x, scale_w):
    x8 = x.astype(jnp.float8_e5m2)
    w8 = w_mat.astype(jnp.float8_e5m2)
    scale = (scale_x[0] * scale_w[0]).reshape(1, 1)

    def body(x_ref, w_ref, s_ref, out_ref, xg, wg, acc,
             xs_send, xs_recv, ws_send, ws_recv,
             xl_send, xl_recv, wl_send, wl_recv, out_sems):
        my = lax.axis_index("i")
        right = lax.rem(my + 1, N_DEV)
        left = lax.rem(my + N_DEV - 1, N_DEV)

        barrier = pltpu.get_barrier_semaphore()
        for nbr in (left, right):
            pl.semaphore_signal(barrier, inc=1, device_id=(nbr,),
                                device_id_type=pl.DeviceIdType.MESH)
        pl.semaphore_wait(barrier, 2)

        xg[:, pl.ds(my * KS, KS)] = x_ref[...]
        wg[pl.ds(my * KS, KS), :] = w_ref[...]

        def mk(ref, row_major, s, ssem, rsem, dst):
            if row_major:
                src = ref.at[pl.ds(s * KS, KS), :]
            else:
                src = ref.at[:, pl.ds(s * KS, KS)]
            return pltpu.make_async_remote_copy(
                src_ref=src, dst_ref=src,
                send_sem=ssem, recv_sem=rsem,
                device_id=(dst,), device_id_type=pl.DeviceIdType.MESH)

        def hop(h, carry):
            sR = lax.rem(my - h + N_DEV, N_DEV)
            rxR = mk(xg, False, sR, xs_send.at[h], xs_recv.at[h], right)
            rwR = mk(wg, True, sR, ws_send.at[h], ws_recv.at[h], right)
            rxR.start()
            rwR.start()

            @pl.when(h < H_LEFT)
            def _():
                sL = lax.rem(my + h, N_DEV)
                rxL = mk(xg, False, sL, xl_send.at[h], xl_recv.at[h], left)
                rwL = mk(wg, True, sL, wl_send.at[h], wl_recv.at[h], left)
                rxL.start()
                rwL.start()
                rxL.wait()
                rwL.wait()

            rxR.wait()
            rwR.wait()
            return carry

        lax.fori_loop(0, H_RIGHT, hop, 0)

        sc = s_ref[0, 0]

        def out_cp(t, b):
            return pltpu.make_async_copy(
                acc.at[b], out_ref.at[:, pl.ds(t * NT, NT)], out_sems.at[b])

        def tile(t, carry):
            b = lax.rem(t, 2)
            w_t = wg[:, pl.ds(t * NT, NT)]

            @pl.when(t >= 2)
            def _():
                out_cp(t - 2, b).wait()

            def mtile(m, carry2):
                a = jnp.dot(xg[pl.ds(m * MT, MT), :], w_t,
                            preferred_element_type=jnp.float32)
                y = a * sc
                acc[b, pl.ds(m * MT, MT), :] = \
                    y * (0.5 + 0.5 * jnp.tanh(0.5 * y))
                return carry2

            lax.fori_loop(0, M_TILES, mtile, 0)
            out_cp(t, b).start()
            return carry

        lax.fori_loop(0, N_TILES, tile, 0)
        out_cp(N_TILES - 2, 0).wait()
        out_cp(N_TILES - 1, 1).wait()

    return pl.pallas_call(
        body,
        out_shape=jax.ShapeDtypeStruct((M, N), jnp.float32),
        in_specs=[
            pl.BlockSpec(memory_space=pltpu.VMEM),
            pl.BlockSpec(memory_space=pltpu.VMEM),
            pl.BlockSpec(memory_space=pltpu.SMEM),
        ],
        out_specs=pl.BlockSpec(memory_space=pltpu.MemorySpace.HBM),
        scratch_shapes=[
            pltpu.VMEM((M, K), jnp.float8_e5m2),
            pltpu.VMEM((K, N), jnp.float8_e5m2),
            pltpu.VMEM((2, M, NT), jnp.float32),
            pltpu.SemaphoreType.DMA((H_RIGHT,)),
            pltpu.SemaphoreType.DMA((H_RIGHT,)),
            pltpu.SemaphoreType.DMA((H_RIGHT,)),
            pltpu.SemaphoreType.DMA((H_RIGHT,)),
            pltpu.SemaphoreType.DMA((H_LEFT,)),
            pltpu.SemaphoreType.DMA((H_LEFT,)),
            pltpu.SemaphoreType.DMA((H_LEFT,)),
            pltpu.SemaphoreType.DMA((H_LEFT,)),
            pltpu.SemaphoreType.DMA((2,)),
        ],
        compiler_params=pltpu.CompilerParams(
            collective_id=0,
            vmem_limit_bytes=100 * 1024 * 1024,
        ),
    )(x8, w8, scale)


# device time: 671973 ns/iter; 1.4560x vs baseline; 1.3330x over previous
import jax
import jax.numpy as jnp
from jax import lax
from jax.experimental import pallas as pl
from jax.experimental.pallas import tpu as pltpu

N_DEV = 32
M, K, N = 4096, 4096, 8192
KS = K // N_DEV
NT = 256
N_TILES = N // NT
MT = 512
M_TILES = M // MT
PLANE = 8
NZ = 4
DO_GATHER = True
DO_COMPUTE = True


def kernel(x, w_mat, scale_x, scale_w):
    x8 = x.astype(jnp.float8_e5m2)
    w8 = w_mat.astype(jnp.float8_e5m2)
    scale = (scale_x[0] * scale_w[0]).reshape(1, 1)

    my = lax.axis_index("i")
    zc = my // PLANE
    pos = my % PLANE
    NEXT_POS = jnp.array([1, 2, 5, 0, 3, 6, 7, 4], jnp.int32)
    PREV_POS = jnp.array([3, 0, 1, 4, 7, 2, 5, 6], jnp.int32)
    CI_OF_POS = jnp.array([0, 1, 2, 7, 6, 3, 4, 5], jnp.int32)
    params = jnp.stack([
        PLANE * zc + NEXT_POS[pos],
        PLANE * zc + PREV_POS[pos],
        jnp.clip(my + PLANE, 0, N_DEV - 1),
        jnp.clip(my - PLANE, 0, N_DEV - 1),
        zc,
        pos,
        CI_OF_POS[pos],
    ]).astype(jnp.int32)
    order = jnp.array([0, 1, 2, 5, 6, 7, 4, 3], jnp.int32)

    def body(x_ref, w_ref, s_ref, p_ref, o_ref, out_ref, xg, wg, acc,
             xzd_s, xzd_r, wzd_s, wzd_r, xzu_s, xzu_r, wzu_s, wzu_r,
             xpr_s, xpr_r, wpr_s, wpr_r, xpl_s, xpl_r, wpl_s, wpl_r,
             out_sems):
        nxt = p_ref[0]
        prv = p_ref[1]
        up = p_ref[2]
        dn = p_ref[3]
        z = p_ref[4]
        pos = p_ref[5]
        ci = p_ref[6]

        barrier = pltpu.get_barrier_semaphore()
        for nbr in (nxt, prv):
            pl.semaphore_signal(barrier, inc=1, device_id=(nbr,),
                                device_id_type=pl.DeviceIdType.MESH)

        @pl.when(z < NZ - 1)
        def _():
            pl.semaphore_signal(barrier, inc=1, device_id=(up,),
                                device_id_type=pl.DeviceIdType.MESH)

        @pl.when(z > 0)
        def _():
            pl.semaphore_signal(barrier, inc=1, device_id=(dn,),
                                device_id_type=pl.DeviceIdType.MESH)

        pl.semaphore_wait(barrier, 2)

        @pl.when(z < NZ - 1)
        def _():
            pl.semaphore_wait(barrier, 1)

        @pl.when(z > 0)
        def _():
            pl.semaphore_wait(barrier, 1)

        mi = PLANE * z + pos
        xg[:, pl.ds(mi * KS, KS)] = x_ref[...]
        wg[pl.ds(mi * KS, KS), :] = w_ref[...]

        def mk(ref, row_major, j, ssem, rsem, dst):
            if row_major:
                src = ref.at[pl.ds(j * KS, KS), :]
            else:
                src = ref.at[:, pl.ds(j * KS, KS)]
            return pltpu.make_async_remote_copy(
                src_ref=src, dst_ref=src,
                send_sem=ssem, recv_sem=rsem,
                device_id=(dst,), device_id_type=pl.DeviceIdType.MESH)

        if DO_GATHER:
            for r in range(NZ - 1):
                snd_dn = (z >= 1) & (z + r <= NZ - 1)
                snd_up = (z <= NZ - 2) & (z >= r)
                rcv_dn = z + 1 + r <= NZ - 1
                rcv_up = z >= r + 1

                @pl.when(snd_dn)
                def _(r=r):
                    j = PLANE * (z + r) + pos
                    mk(xg, False, j, xzd_s.at[r], xzd_r.at[r], dn).start()
                    mk(wg, True, j, wzd_s.at[r], wzd_r.at[r], dn).start()

                @pl.when(snd_up)
                def _(r=r):
                    j = PLANE * (z - r) + pos
                    mk(xg, False, j, xzu_s.at[r], xzu_r.at[r], up).start()
                    mk(wg, True, j, wzu_s.at[r], wzu_r.at[r], up).start()

                @pl.when(rcv_dn)
                def _(r=r):
                    j = PLANE * (z + 1 + r) + pos
                    mk(xg, False, j, xzd_s.at[r], xzd_r.at[r], dn).wait_recv()
                    mk(wg, True, j, wzd_s.at[r], wzd_r.at[r], dn).wait_recv()

                @pl.when(rcv_up)
                def _(r=r):
                    j = PLANE * (z - 1 - r) + pos
                    mk(xg, False, j, xzu_s.at[r], xzu_r.at[r], up).wait_recv()
                    mk(wg, True, j, wzu_s.at[r], wzu_r.at[r], up).wait_recv()

                @pl.when(snd_dn)
                def _(r=r):
                    j = PLANE * (z + r) + pos
                    mk(xg, False, j, xzd_s.at[r], xzd_r.at[r], dn).wait_send()
                    mk(wg, True, j, wzd_s.at[r], wzd_r.at[r], dn).wait_send()

                @pl.when(snd_up)
                def _(r=r):
                    j = PLANE * (z - r) + pos
                    mk(xg, False, j, xzu_s.at[r], xzu_r.at[r], up).wait_send()
                    mk(wg, True, j, wzu_s.at[r], wzu_r.at[r], up).wait_send()

            for h in range(PLANE // 2):
                gs = o_ref[lax.rem(ci - h + PLANE, PLANE)]
                rdesc = []
                for k in range(NZ):
                    j = PLANE * k + gs
                    rx = mk(xg, False, j, xpr_s.at[h, k], xpr_r.at[h, k], nxt)
                    rw = mk(wg, True, j, wpr_s.at[h, k], wpr_r.at[h, k], nxt)
                    rx.start()
                    rw.start()
                    rdesc += [rx, rw]
                if h < PLANE // 2 - 1:
                    gl = o_ref[lax.rem(ci + h, PLANE)]
                    ldesc = []
                    for k in range(NZ):
                        j = PLANE * k + gl
                        lx = mk(xg, False, j,
                                xpl_s.at[h, k], xpl_r.at[h, k], prv)
                        lw = mk(wg, True, j,
                                wpl_s.at[h, k], wpl_r.at[h, k], prv)
                        lx.start()
                        lw.start()
                        ldesc += [lx, lw]
                    for d in ldesc:
                        d.wait()
                for d in rdesc:
                    d.wait()

        sc = s_ref[0, 0]

        def out_cp(t, b):
            return pltpu.make_async_copy(
                acc.at[b], out_ref.at[:, pl.ds(t * NT, NT)], out_sems.at[b])

        def tile(t, carry):
            b = lax.rem(t, 2)
            w_t = wg[:, pl.ds(t * NT, NT)]

            @pl.when(t >= 2)
            def _():
                out_cp(t - 2, b).wait()

            def mtile(m, carry2):
                a = jnp.dot(xg[pl.ds(m * MT, MT), :], w_t,
                            preferred_element_type=jnp.float32)
                y = a * sc
                acc[b, pl.ds(m * MT, MT), :] = \
                    y * (0.5 + 0.5 * jnp.tanh(0.5 * y))
                return carry2

            lax.fori_loop(0, M_TILES, mtile, 0)
            out_cp(t, b).start()
            return carry

        if DO_COMPUTE:
            lax.fori_loop(0, N_TILES, tile, 0)
            out_cp(N_TILES - 2, 0).wait()
            out_cp(N_TILES - 1, 1).wait()

    return pl.pallas_call(
        body,
        out_shape=jax.ShapeDtypeStruct((M, N), jnp.float32),
        in_specs=[
            pl.BlockSpec(memory_space=pltpu.VMEM),
            pl.BlockSpec(memory_space=pltpu.VMEM),
            pl.BlockSpec(memory_space=pltpu.SMEM),
            pl.BlockSpec(memory_space=pltpu.SMEM),
            pl.BlockSpec(memory_space=pltpu.SMEM),
        ],
        out_specs=pl.BlockSpec(memory_space=pltpu.MemorySpace.HBM),
        scratch_shapes=[
            pltpu.VMEM((M, K), jnp.float8_e5m2),
            pltpu.VMEM((K, N), jnp.float8_e5m2),
            pltpu.VMEM((2, M, NT), jnp.float32),
            pltpu.SemaphoreType.DMA((NZ - 1,)),
            pltpu.SemaphoreType.DMA((NZ - 1,)),
            pltpu.SemaphoreType.DMA((NZ - 1,)),
            pltpu.SemaphoreType.DMA((NZ - 1,)),
            pltpu.SemaphoreType.DMA((NZ - 1,)),
            pltpu.SemaphoreType.DMA((NZ - 1,)),
            pltpu.SemaphoreType.DMA((NZ - 1,)),
            pltpu.SemaphoreType.DMA((NZ - 1,)),
            pltpu.SemaphoreType.DMA((PLANE // 2, NZ)),
            pltpu.SemaphoreType.DMA((PLANE // 2, NZ)),
            pltpu.SemaphoreType.DMA((PLANE // 2, NZ)),
            pltpu.SemaphoreType.DMA((PLANE // 2, NZ)),
            pltpu.SemaphoreType.DMA((PLANE // 2 - 1, NZ)),
            pltpu.SemaphoreType.DMA((PLANE // 2 - 1, NZ)),
            pltpu.SemaphoreType.DMA((PLANE // 2 - 1, NZ)),
            pltpu.SemaphoreType.DMA((PLANE // 2 - 1, NZ)),
            pltpu.SemaphoreType.DMA((2,)),
        ],
        compiler_params=pltpu.CompilerParams(
            collective_id=0,
            vmem_limit_bytes=100 * 1024 * 1024,
        ),
    )(x8, w8, scale, params, order)


# device time: 671781 ns/iter; 1.4564x vs baseline; 1.0003x over previous
import jax
import jax.numpy as jnp
from jax import lax
from jax.experimental import pallas as pl
from jax.experimental.pallas import tpu as pltpu

N_DEV = 32
M, K, N = 4096, 4096, 8192
KS = K // N_DEV
NT = 256
N_TILES = N // NT
MT = 512
M_TILES = M // MT
PLANE = 8
NZ = 4
DO_GATHER = True
DO_COMPUTE = True


def kernel(x, w_mat, scale_x, scale_w):
    x8 = x.astype(jnp.float8_e4m3fn)
    w8 = w_mat.astype(jnp.float8_e4m3fn)
    scale = (scale_x[0] * scale_w[0]).reshape(1, 1)

    my = lax.axis_index("i")
    zc = my // PLANE
    pos = my % PLANE
    NEXT_POS = jnp.array([1, 2, 5, 0, 3, 6, 7, 4], jnp.int32)
    PREV_POS = jnp.array([3, 0, 1, 4, 7, 2, 5, 6], jnp.int32)
    CI_OF_POS = jnp.array([0, 1, 2, 7, 6, 3, 4, 5], jnp.int32)
    params = jnp.stack([
        PLANE * zc + NEXT_POS[pos],
        PLANE * zc + PREV_POS[pos],
        jnp.clip(my + PLANE, 0, N_DEV - 1),
        jnp.clip(my - PLANE, 0, N_DEV - 1),
        zc,
        pos,
        CI_OF_POS[pos],
    ]).astype(jnp.int32)
    order = jnp.array([0, 1, 2, 5, 6, 7, 4, 3], jnp.int32)

    def body(x_ref, w_ref, s_ref, p_ref, o_ref, out_ref, xg, wg, acc,
             xzd_s, xzd_r, wzd_s, wzd_r, xzu_s, xzu_r, wzu_s, wzu_r,
             xpr_s, xpr_r, wpr_s, wpr_r, xpl_s, xpl_r, wpl_s, wpl_r,
             out_sems):
        nxt = p_ref[0]
        prv = p_ref[1]
        up = p_ref[2]
        dn = p_ref[3]
        z = p_ref[4]
        pos = p_ref[5]
        ci = p_ref[6]

        barrier = pltpu.get_barrier_semaphore()
        for nbr in (nxt, prv):
            pl.semaphore_signal(barrier, inc=1, device_id=(nbr,),
                                device_id_type=pl.DeviceIdType.MESH)

        @pl.when(z < NZ - 1)
        def _():
            pl.semaphore_signal(barrier, inc=1, device_id=(up,),
                                device_id_type=pl.DeviceIdType.MESH)

        @pl.when(z > 0)
        def _():
            pl.semaphore_signal(barrier, inc=1, device_id=(dn,),
                                device_id_type=pl.DeviceIdType.MESH)

        pl.semaphore_wait(barrier, 2)

        @pl.when(z < NZ - 1)
        def _():
            pl.semaphore_wait(barrier, 1)

        @pl.when(z > 0)
        def _():
            pl.semaphore_wait(barrier, 1)

        mi = PLANE * z + pos
        xg[:, pl.ds(mi * KS, KS)] = x_ref[...]
        wg[pl.ds(mi * KS, KS), :] = w_ref[...]

        def mk(ref, row_major, j, ssem, rsem, dst):
            if row_major:
                src = ref.at[pl.ds(j * KS, KS), :]
            else:
                src = ref.at[:, pl.ds(j * KS, KS)]
            return pltpu.make_async_remote_copy(
                src_ref=src, dst_ref=src,
                send_sem=ssem, recv_sem=rsem,
                device_id=(dst,), device_id_type=pl.DeviceIdType.MESH)

        if DO_GATHER:
            for r in range(NZ - 1):
                snd_dn = (z >= 1) & (z + r <= NZ - 1)
                snd_up = (z <= NZ - 2) & (z >= r)
                rcv_dn = z + 1 + r <= NZ - 1
                rcv_up = z >= r + 1

                @pl.when(snd_dn)
                def _(r=r):
                    j = PLANE * (z + r) + pos
                    mk(xg, False, j, xzd_s.at[r], xzd_r.at[r], dn).start()
                    mk(wg, True, j, wzd_s.at[r], wzd_r.at[r], dn).start()

                @pl.when(snd_up)
                def _(r=r):
                    j = PLANE * (z - r) + pos
                    mk(xg, False, j, xzu_s.at[r], xzu_r.at[r], up).start()
                    mk(wg, True, j, wzu_s.at[r], wzu_r.at[r], up).start()

                @pl.when(rcv_dn)
                def _(r=r):
                    j = PLANE * (z + 1 + r) + pos
                    mk(xg, False, j, xzd_s.at[r], xzd_r.at[r], dn).wait_recv()
                    mk(wg, True, j, wzd_s.at[r], wzd_r.at[r], dn).wait_recv()

                @pl.when(rcv_up)
                def _(r=r):
                    j = PLANE * (z - 1 - r) + pos
                    mk(xg, False, j, xzu_s.at[r], xzu_r.at[r], up).wait_recv()
                    mk(wg, True, j, wzu_s.at[r], wzu_r.at[r], up).wait_recv()

                @pl.when(snd_dn)
                def _(r=r):
                    j = PLANE * (z + r) + pos
                    mk(xg, False, j, xzd_s.at[r], xzd_r.at[r], dn).wait_send()
                    mk(wg, True, j, wzd_s.at[r], wzd_r.at[r], dn).wait_send()

                @pl.when(snd_up)
                def _(r=r):
                    j = PLANE * (z - r) + pos
                    mk(xg, False, j, xzu_s.at[r], xzu_r.at[r], up).wait_send()
                    mk(wg, True, j, wzu_s.at[r], wzu_r.at[r], up).wait_send()

            for h in range(PLANE // 2):
                gs = o_ref[lax.rem(ci - h + PLANE, PLANE)]
                rdesc = []
                for k in range(NZ):
                    j = PLANE * k + gs
                    rx = mk(xg, False, j, xpr_s.at[h, k], xpr_r.at[h, k], nxt)
                    rw = mk(wg, True, j, wpr_s.at[h, k], wpr_r.at[h, k], nxt)
                    rx.start()
                    rw.start()
                    rdesc += [rx, rw]
                if h < PLANE // 2 - 1:
                    gl = o_ref[lax.rem(ci + h, PLANE)]
                    ldesc = []
                    for k in range(NZ):
                        j = PLANE * k + gl
                        lx = mk(xg, False, j,
                                xpl_s.at[h, k], xpl_r.at[h, k], prv)
                        lw = mk(wg, True, j,
                                wpl_s.at[h, k], wpl_r.at[h, k], prv)
                        lx.start()
                        lw.start()
                        ldesc += [lx, lw]
                    for d in ldesc:
                        d.wait()
                for d in rdesc:
                    d.wait()

        sc = s_ref[0, 0]

        def out_cp(t, b):
            return pltpu.make_async_copy(
                acc.at[b], out_ref.at[:, pl.ds(t * NT, NT)], out_sems.at[b])

        def tile(t, carry):
            b = lax.rem(t, 2)
            w_t = wg[:, pl.ds(t * NT, NT)]

            @pl.when(t >= 2)
            def _():
                out_cp(t - 2, b).wait()

            def mtile(m, carry2):
                a = jnp.dot(xg[pl.ds(m * MT, MT), :], w_t,
                            preferred_element_type=jnp.float32)
                y = a * sc
                acc[b, pl.ds(m * MT, MT), :] = \
                    y * (0.5 + 0.5 * jnp.tanh(0.5 * y))
                return carry2

            lax.fori_loop(0, M_TILES, mtile, 0)
            out_cp(t, b).start()
            return carry

        if DO_COMPUTE:
            lax.fori_loop(0, N_TILES, tile, 0)
            out_cp(N_TILES - 2, 0).wait()
            out_cp(N_TILES - 1, 1).wait()

    return pl.pallas_call(
        body,
        out_shape=jax.ShapeDtypeStruct((M, N), jnp.float32),
        in_specs=[
            pl.BlockSpec(memory_space=pltpu.VMEM),
            pl.BlockSpec(memory_space=pltpu.VMEM),
            pl.BlockSpec(memory_space=pltpu.SMEM),
            pl.BlockSpec(memory_space=pltpu.SMEM),
            pl.BlockSpec(memory_space=pltpu.SMEM),
        ],
        out_specs=pl.BlockSpec(memory_space=pltpu.MemorySpace.HBM),
        scratch_shapes=[
            pltpu.VMEM((M, K), jnp.float8_e4m3fn),
            pltpu.VMEM((K, N), jnp.float8_e4m3fn),
            pltpu.VMEM((2, M, NT), jnp.float32),
            pltpu.SemaphoreType.DMA((NZ - 1,)),
            pltpu.SemaphoreType.DMA((NZ - 1,)),
            pltpu.SemaphoreType.DMA((NZ - 1,)),
            pltpu.SemaphoreType.DMA((NZ - 1,)),
            pltpu.SemaphoreType.DMA((NZ - 1,)),
            pltpu.SemaphoreType.DMA((NZ - 1,)),
            pltpu.SemaphoreType.DMA((NZ - 1,)),
            pltpu.SemaphoreType.DMA((NZ - 1,)),
            pltpu.SemaphoreType.DMA((PLANE // 2, NZ)),
            pltpu.SemaphoreType.DMA((PLANE // 2, NZ)),
            pltpu.SemaphoreType.DMA((PLANE // 2, NZ)),
            pltpu.SemaphoreType.DMA((PLANE // 2, NZ)),
            pltpu.SemaphoreType.DMA((PLANE // 2 - 1, NZ)),
            pltpu.SemaphoreType.DMA((PLANE // 2 - 1, NZ)),
            pltpu.SemaphoreType.DMA((PLANE // 2 - 1, NZ)),
            pltpu.SemaphoreType.DMA((PLANE // 2 - 1, NZ)),
            pltpu.SemaphoreType.DMA((2,)),
        ],
        compiler_params=pltpu.CompilerParams(
            collective_id=0,
            vmem_limit_bytes=100 * 1024 * 1024,
        ),
    )(x8, w8, scale, params, order)
